# Initial kernel scaffold; baseline (speedup 1.0000x reference)
import jax
import jax.numpy as jnp
from jax.experimental import pallas as pl


def kernel(input, weight, absmax, code, adapter_emb, adapter_W):
    raise NotImplementedError("write your pallas kernel here")


# speedup vs baseline: 183.0750x; 183.0750x over previous
import functools

import jax
import jax.numpy as jnp
from jax import lax
from jax.experimental import pallas as pl
from jax.experimental.pallas import tpu as pltpu, tpu_sc as plsc

V = 100000
D = 128
A = 64
B = 4096
L = 50
N = B * L
NB = (V * D) // 4096
NB_PAD = 3200

NC = 2
NS = 16
NW = NC * NS

ROWS_PER_W = V // NW
DQ_CHUNK = 125
DQ_ELEMS = DQ_CHUNK * D

TOK_PER_W = N // NW
G_CHUNK = 128
G_STEPS = TOK_PER_W // G_CHUNK


def _adapter_matmul(adapter_emb, adapter_W):
    def body(emb_ref, w_ref, out_ref):
        out_ref[...] = lax.dot_general(
            emb_ref[...], w_ref[...],
            dimension_numbers=(((1,), (1,)), ((), ())),
            preferred_element_type=jnp.float32)

    blk = 800
    return pl.pallas_call(
        body,
        grid=(V // blk,),
        in_specs=[
            pl.BlockSpec((blk, A), lambda i: (i, 0)),
            pl.BlockSpec((D, A), lambda i: (0, 0)),
        ],
        out_specs=pl.BlockSpec((blk, D), lambda i: (i, 0)),
        out_shape=jax.ShapeDtypeStruct((V, D), jnp.float32),
    )(adapter_emb, adapter_W)


def _dequant_table(weight_flat, absmax_pad, code, p_flat):
    mesh = plsc.VectorSubcoreMesh(
        core_axis_name="c", subcore_axis_name="s",
        num_cores=NC, num_subcores=NS)

    @functools.partial(
        pl.kernel,
        out_type=jax.ShapeDtypeStruct((V * D,), jnp.float32),
        mesh=mesh,
        scratch_types=[
            pltpu.VMEM((256,), jnp.float32),
            pltpu.VMEM((NB_PAD,), jnp.float32),
            pltpu.VMEM((DQ_ELEMS,), jnp.int32),
            pltpu.VMEM((DQ_ELEMS,), jnp.float32),
        ],
        compiler_params=pltpu.CompilerParams(needs_layout_passes=False),
    )
    def run(w_hbm, am_hbm, code_hbm, p_hbm, f_hbm, code_v, am_v, w_v, p_v):
        wid = lax.axis_index("s") * NC + lax.axis_index("c")
        pltpu.sync_copy(code_hbm, code_v)
        pltpu.sync_copy(am_hbm, am_v)
        row0 = wid * ROWS_PER_W

        @pl.loop(0, ROWS_PER_W // DQ_CHUNK)
        def chunk_loop(ch):
            base = (row0 + ch * DQ_CHUNK) * D
            pltpu.sync_copy(w_hbm.at[pl.ds(base, DQ_ELEMS)], w_v)
            pltpu.sync_copy(p_hbm.at[pl.ds(base, DQ_ELEMS)], p_v)

            @pl.loop(0, DQ_CHUNK)
            def row_loop(r):
                row = row0 + ch * DQ_CHUNK + r
                am16 = plsc.load_gather(
                    am_v, [jnp.full((16,), row >> 5, jnp.int32)])
                for k in range(D // 16):
                    off = r * D + k * 16
                    w16 = w_v[pl.ds(off, 16)]
                    c16 = plsc.load_gather(code_v, [w16])
                    p_v[pl.ds(off, 16)] = c16 * am16 + p_v[pl.ds(off, 16)]

            pltpu.sync_copy(p_v, f_hbm.at[pl.ds(base, DQ_ELEMS)])

    return run(weight_flat, absmax_pad, code, p_flat)


def _gather_rows(table, idx):
    mesh = plsc.VectorSubcoreMesh(
        core_axis_name="c", subcore_axis_name="s",
        num_cores=NC, num_subcores=NS)

    @functools.partial(
        pl.kernel,
        out_type=jax.ShapeDtypeStruct((N, D), jnp.float32),
        mesh=mesh,
        scratch_types=[
            pltpu.VMEM((G_CHUNK,), jnp.int32),
            pltpu.VMEM((G_CHUNK, D), jnp.float32),
            pltpu.SemaphoreType.DMA,
        ],
        compiler_params=pltpu.CompilerParams(needs_layout_passes=False),
    )
    def run(tab_hbm, idx_hbm, out_hbm, idx_v, rows_v, sem):
        wid = lax.axis_index("s") * NC + lax.axis_index("c")
        tok0 = wid * TOK_PER_W

        @pl.loop(0, G_STEPS)
        def step(g):
            base = tok0 + g * G_CHUNK
            pltpu.sync_copy(idx_hbm.at[pl.ds(base, G_CHUNK)], idx_v)
            pltpu.async_copy(tab_hbm.at[idx_v], rows_v, sem).wait()
            pltpu.sync_copy(rows_v, out_hbm.at[pl.ds(base, G_CHUNK)])

    return run(table, idx)


def kernel(input, weight, absmax, code, adapter_emb, adapter_W):
    idx = input.reshape(N)
    absmax_pad = jnp.pad(absmax, (0, NB_PAD - NB))
    p = _adapter_matmul(adapter_emb, adapter_W)
    f = _dequant_table(weight.reshape(V * D), absmax_pad, code, p.reshape(V * D))
    out = _gather_rows(f.reshape(V, D), idx)
    return out.reshape(B, L, D)


# speedup vs baseline: 184.9192x; 1.0101x over previous
import functools

import jax
import jax.numpy as jnp
from jax import lax
from jax.experimental import pallas as pl
from jax.experimental.pallas import tpu as pltpu, tpu_sc as plsc

V = 100000
D = 128
A = 64
B = 4096
L = 50
N = B * L
NB = (V * D) // 4096
NB_PAD = 3200

NC = 2
NS = 16
NW = NC * NS

DQ_CHUNK = 200
DQ_NCHUNKS = V // DQ_CHUNK

TOK_PER_W = N // NW
G_CHUNK = 128
G_STEPS = TOK_PER_W // G_CHUNK


def _adapter_matmul(adapter_emb, adapter_W):
    def body(emb_ref, w_ref, out_ref):
        out_ref[...] = lax.dot_general(
            emb_ref[...], w_ref[...],
            dimension_numbers=(((1,), (1,)), ((), ())),
            preferred_element_type=jnp.float32)

    blk = 800
    return pl.pallas_call(
        body,
        grid=(V // blk,),
        in_specs=[
            pl.BlockSpec((blk, A), lambda i: (i, 0)),
            pl.BlockSpec((D, A), lambda i: (0, 0)),
        ],
        out_specs=pl.BlockSpec((blk, D), lambda i: (i, 0)),
        out_shape=jax.ShapeDtypeStruct((V, D), jnp.float32),
    )(adapter_emb, adapter_W)


def _dequant_table(weight, absmax_pad, code, p):
    mesh = plsc.VectorSubcoreMesh(
        core_axis_name="c", subcore_axis_name="s",
        num_cores=NC, num_subcores=NS)

    @functools.partial(
        pl.kernel,
        out_type=jax.ShapeDtypeStruct((V, D), jnp.float32),
        mesh=mesh,
        scratch_types=[
            pltpu.VMEM((256,), jnp.float32),
            pltpu.VMEM((NB_PAD,), jnp.float32),
            pltpu.VMEM((DQ_CHUNK, D), jnp.int32),
            pltpu.VMEM((DQ_CHUNK, D), jnp.float32),
        ],
        compiler_params=pltpu.CompilerParams(needs_layout_passes=False),
    )
    def run(w_hbm, am_hbm, code_hbm, p_hbm, f_hbm, code_v, am_v, w_v, p_v):
        wid = lax.axis_index("s") * NC + lax.axis_index("c")
        pltpu.sync_copy(code_hbm, code_v)
        pltpu.sync_copy(am_hbm, am_v)

        @pl.loop(wid, DQ_NCHUNKS, step=NW)
        def chunk_loop(ch):
            base = ch * DQ_CHUNK
            pltpu.sync_copy(w_hbm.at[pl.ds(base, DQ_CHUNK)], w_v)
            pltpu.sync_copy(p_hbm.at[pl.ds(base, DQ_CHUNK)], p_v)

            @pl.loop(0, DQ_CHUNK)
            def row_loop(r):
                row = base + r
                am16 = plsc.load_gather(
                    am_v, [jnp.full((16,), row >> 5, jnp.int32)])
                for k in range(D // 16):
                    w16 = w_v[r, pl.ds(k * 16, 16)]
                    c16 = plsc.load_gather(code_v, [w16])
                    p_v[r, pl.ds(k * 16, 16)] = (
                        c16 * am16 + p_v[r, pl.ds(k * 16, 16)])

            pltpu.sync_copy(p_v, f_hbm.at[pl.ds(base, DQ_CHUNK)])

    return run(weight, absmax_pad, code, p)


def _gather_rows(table, idx):
    mesh = plsc.VectorSubcoreMesh(
        core_axis_name="c", subcore_axis_name="s",
        num_cores=NC, num_subcores=NS)

    @functools.partial(
        pl.kernel,
        out_type=jax.ShapeDtypeStruct((N, D), jnp.float32),
        mesh=mesh,
        scratch_types=[
            pltpu.VMEM((G_CHUNK,), jnp.int32),
            pltpu.VMEM((G_CHUNK, D), jnp.float32),
            pltpu.SemaphoreType.DMA,
        ],
        compiler_params=pltpu.CompilerParams(needs_layout_passes=False),
    )
    def run(tab_hbm, idx_hbm, out_hbm, idx_v, rows_v, sem):
        wid = lax.axis_index("s") * NC + lax.axis_index("c")
        tok0 = wid * TOK_PER_W

        @pl.loop(0, G_STEPS)
        def step(g):
            base = tok0 + g * G_CHUNK
            pltpu.sync_copy(idx_hbm.at[pl.ds(base, G_CHUNK)], idx_v)
            pltpu.async_copy(tab_hbm.at[idx_v], rows_v, sem).wait()
            pltpu.sync_copy(rows_v, out_hbm.at[pl.ds(base, G_CHUNK)])

    return run(table, idx)


def kernel(input, weight, absmax, code, adapter_emb, adapter_W):
    idx = input.reshape(N)
    absmax_pad = jnp.pad(absmax, (0, NB_PAD - NB))
    p = _adapter_matmul(adapter_emb, adapter_W)
    f = _dequant_table(weight, absmax_pad, code, p)
    out = _gather_rows(f, idx)
    return out.reshape(B, L, D)


# speedup vs baseline: 305.5690x; 1.6524x over previous
import functools

import jax
import jax.numpy as jnp
from jax import lax
from jax.experimental import pallas as pl
from jax.experimental.pallas import tpu as pltpu, tpu_sc as plsc

V = 100000
D = 128
A = 64
B = 4096
L = 50
N = B * L
NB = (V * D) // 4096
NB_PAD = 3200

NC = 2
NS = 16
NW = NC * NS

DQ_CHUNK = 40
DQ_NCHUNKS = V // DQ_CHUNK

TOK_PER_W = N // NW
G_CHUNK = 128
G_STEPS = TOK_PER_W // G_CHUNK


def _adapter_matmul(emb_t, w_t):
    def body(emb_ref, w_ref, out_ref):
        out_ref[...] = lax.dot_general(
            emb_ref[...], w_ref[...],
            dimension_numbers=(((0,), (0,)), ((), ())),
            preferred_element_type=jnp.float32)

    blk = 2048
    return pl.pallas_call(
        body,
        grid=((V + blk - 1) // blk,),
        in_specs=[
            pl.BlockSpec((A, blk), lambda i: (0, i)),
            pl.BlockSpec((A, D), lambda i: (0, 0)),
        ],
        out_specs=pl.BlockSpec((blk, D), lambda i: (i, 0)),
        out_shape=jax.ShapeDtypeStruct((V, D), jnp.float32),
    )(emb_t, w_t)


def _dequant_table(weight, absmax_pad, code, p):
    mesh = plsc.VectorSubcoreMesh(
        core_axis_name="c", subcore_axis_name="s",
        num_cores=NC, num_subcores=NS)

    @functools.partial(
        pl.kernel,
        out_type=jax.ShapeDtypeStruct((V, D), jnp.float32),
        mesh=mesh,
        scratch_types=[
            pltpu.VMEM((256,), jnp.float32),
            pltpu.VMEM((NB_PAD,), jnp.float32),
            pltpu.VMEM((2, DQ_CHUNK, D), jnp.int32),
            pltpu.VMEM((2, DQ_CHUNK, D), jnp.float32),
            pltpu.VMEM((2, DQ_CHUNK, D), jnp.float32),
            pltpu.SemaphoreType.DMA((2,)),
            pltpu.SemaphoreType.DMA((2,)),
            pltpu.SemaphoreType.DMA((2,)),
        ],
        compiler_params=pltpu.CompilerParams(needs_layout_passes=False),
    )
    def run(w_hbm, am_hbm, code_hbm, p_hbm, f_hbm,
            code_v, am_v, w_v, p_v, o_v, sem_w, sem_p, sem_o):
        wid = lax.axis_index("s") * NC + lax.axis_index("c")
        pltpu.sync_copy(code_hbm, code_v)
        pltpu.sync_copy(am_hbm, am_v)
        nj = (DQ_NCHUNKS - wid + NW - 1) // NW

        def start_in(ch, s):
            base = ch * DQ_CHUNK
            pltpu.async_copy(
                w_hbm.at[pl.ds(base, DQ_CHUNK)], w_v.at[s], sem_w.at[s])
            pltpu.async_copy(
                p_hbm.at[pl.ds(base, DQ_CHUNK)], p_v.at[s], sem_p.at[s])

        def wait_in(s):
            pltpu.make_async_copy(
                w_hbm.at[pl.ds(0, DQ_CHUNK)], w_v.at[s], sem_w.at[s]).wait()
            pltpu.make_async_copy(
                p_hbm.at[pl.ds(0, DQ_CHUNK)], p_v.at[s], sem_p.at[s]).wait()

        def start_out(ch, s):
            pltpu.async_copy(
                o_v.at[s], f_hbm.at[pl.ds(ch * DQ_CHUNK, DQ_CHUNK)],
                sem_o.at[s])

        def wait_out(s):
            pltpu.make_async_copy(
                o_v.at[s], f_hbm.at[pl.ds(0, DQ_CHUNK)], sem_o.at[s]).wait()

        start_in(wid, 0)

        @pl.loop(0, nj)
        def chunk_loop(j):
            ch = wid + j * NW
            s = j % 2

            @pl.when(j + 1 < nj)
            def _():
                start_in(ch + NW, 1 - s)

            wait_in(s)

            @pl.when(j >= 2)
            def _():
                wait_out(s)

            @pl.loop(0, DQ_CHUNK)
            def row_loop(r):
                row = ch * DQ_CHUNK + r
                am16 = plsc.load_gather(
                    am_v, [jnp.full((16,), row >> 5, jnp.int32)])
                for k in range(D // 16):
                    w16 = w_v[s, r, pl.ds(k * 16, 16)]
                    c16 = plsc.load_gather(code_v, [w16])
                    o_v[s, r, pl.ds(k * 16, 16)] = (
                        c16 * am16 + p_v[s, r, pl.ds(k * 16, 16)])

            start_out(ch, s)

        wait_out(nj % 2)
        wait_out((nj + 1) % 2)

    return run(weight, absmax_pad, code, p)


def _gather_rows(table, idx):
    mesh = plsc.VectorSubcoreMesh(
        core_axis_name="c", subcore_axis_name="s",
        num_cores=NC, num_subcores=NS)

    @functools.partial(
        pl.kernel,
        out_type=jax.ShapeDtypeStruct((N, D), jnp.float32),
        mesh=mesh,
        scratch_types=[
            pltpu.VMEM((TOK_PER_W,), jnp.int32),
            pltpu.VMEM((3, G_CHUNK, D), jnp.float32),
            pltpu.SemaphoreType.DMA((3,)),
            pltpu.SemaphoreType.DMA((3,)),
        ],
        compiler_params=pltpu.CompilerParams(needs_layout_passes=False),
    )
    def run(tab_hbm, idx_hbm, out_hbm, idx_v, rows_v, sem_g, sem_o):
        wid = lax.axis_index("s") * NC + lax.axis_index("c")
        tok0 = wid * TOK_PER_W
        pltpu.sync_copy(idx_hbm.at[pl.ds(tok0, TOK_PER_W)], idx_v)

        def start_gather(g, s):
            pltpu.async_copy(
                tab_hbm.at[idx_v.at[pl.ds(g * G_CHUNK, G_CHUNK)]],
                rows_v.at[s], sem_g.at[s])

        def wait_gather(s):
            pltpu.make_async_copy(
                tab_hbm.at[idx_v.at[pl.ds(0, G_CHUNK)]],
                rows_v.at[s], sem_g.at[s]).wait()

        def start_out(g, s):
            pltpu.async_copy(
                rows_v.at[s],
                out_hbm.at[pl.ds(tok0 + g * G_CHUNK, G_CHUNK)], sem_o.at[s])

        def wait_out(s):
            pltpu.make_async_copy(
                rows_v.at[s], out_hbm.at[pl.ds(0, G_CHUNK)],
                sem_o.at[s]).wait()

        start_gather(0, 0)

        @pl.loop(0, G_STEPS)
        def step(g):
            s = g % 3

            @pl.when(g + 1 < G_STEPS)
            def _():
                @pl.when(g >= 2)
                def _():
                    wait_out((g + 1) % 3)
                start_gather(g + 1, (g + 1) % 3)

            wait_gather(s)
            start_out(g, s)

        wait_out(0)
        wait_out(1)
        wait_out(2)

    return run(table, idx)


def kernel(input, weight, absmax, code, adapter_emb, adapter_W):
    idx_t = input.T.reshape(N)
    absmax_pad = jnp.pad(absmax, (0, NB_PAD - NB))
    p = _adapter_matmul(adapter_emb.T, adapter_W.T)
    f = _dequant_table(weight, absmax_pad, code, p)
    out_t = _gather_rows(f, idx_t)
    return out_t.reshape(L, B, D).transpose(1, 0, 2)
